# baseline (speedup 1.0000x reference)
import functools

import jax
import jax.numpy as jnp
from jax import lax
from jax.experimental import pallas as pl
from jax.experimental.pallas import tpu as pltpu
from jax.experimental.pallas import tpu_sc as plsc

VOCAB = 1000000
EMB = 128
WINDOW = 5
HIDDEN = 256
N_TAGS = 50
BATCH = 16384

N_IDX = BATCH * WINDOW
NW = 32
B_PER_W = N_IDX // NW
CHUNK = 160
N_CHUNKS = B_PER_W // CHUNK
NBUF = 4
BM = 2048


def _sc_gather_body(n_chunks, table_hbm, idx_hbm, out_hbm, idx_v, *bufs):
    b_per_w = n_chunks * CHUNK
    rows = bufs[:NBUF]
    sems = bufs[NBUF:]
    c = lax.axis_index("c")
    s = lax.axis_index("s")
    wid = s * 2 + c
    base = wid * b_per_w
    pltpu.sync_copy(idx_hbm.at[pl.ds(base, b_per_w)], idx_v)
    descs = [None] * NBUF
    for k in range(NBUF - 1):
        descs[k] = pltpu.async_copy(
            table_hbm.at[idx_v.at[pl.ds(k * CHUNK, CHUNK)]], rows[k], sems[k]
        )
    for k in range(n_chunks):
        b = k % NBUF
        kn = k + NBUF - 1
        if kn < n_chunks:
            bn = kn % NBUF
            descs[bn] = pltpu.async_copy(
                table_hbm.at[idx_v.at[pl.ds(kn * CHUNK, CHUNK)]],
                rows[bn],
                sems[bn],
            )
        descs[b].wait()
        pltpu.sync_copy(rows[b], out_hbm.at[pl.ds(base + k * CHUNK, CHUNK)])


def _make_sc_gather(bh):
    n_idx = bh * WINDOW
    b_per_w = n_idx // NW
    n_chunks = b_per_w // CHUNK
    assert b_per_w % CHUNK == 0
    mesh = plsc.VectorSubcoreMesh(core_axis_name="c", subcore_axis_name="s")
    run = pl.kernel(
        functools.partial(_sc_gather_body, n_chunks),
        mesh=mesh,
        out_type=jax.ShapeDtypeStruct((n_idx, EMB), jnp.float32),
        scratch_types=(
            [pltpu.VMEM((b_per_w,), jnp.int32)]
            + [pltpu.VMEM((CHUNK, EMB), jnp.float32) for _ in range(NBUF)]
            + [pltpu.SemaphoreType.DMA for _ in range(NBUF)]
        ),
    )
    return run


SPLITS = (10240, 6144)
_SC_GATHERS = {bh: _make_sc_gather(bh) for bh in set(SPLITS)}


def _mlp_body(rows_ref, w1_ref, b1_ref, w2_ref, b2_ref, out_ref):
    acc = b1_ref[...] + jnp.dot(
        rows_ref[0], w1_ref[0], preferred_element_type=jnp.float32
    )
    for w in range(1, WINDOW):
        acc = acc + jnp.dot(
            rows_ref[w], w1_ref[w], preferred_element_type=jnp.float32
        )
    h = jnp.tanh(acc)
    out = jnp.dot(h, w2_ref[...], preferred_element_type=jnp.float32) + b2_ref[...]
    out_ref[...] = out[:, :N_TAGS]


def _mlp(bh, rows3, W13, b1, W2p, b2p):
    return pl.pallas_call(
        _mlp_body,
        grid=(bh // BM,),
        in_specs=[
            pl.BlockSpec((WINDOW, BM, EMB), lambda i: (0, i, 0)),
            pl.BlockSpec((WINDOW, EMB, HIDDEN), lambda i: (0, 0, 0)),
            pl.BlockSpec((1, HIDDEN), lambda i: (0, 0)),
            pl.BlockSpec((HIDDEN, 128), lambda i: (0, 0)),
            pl.BlockSpec((1, 128), lambda i: (0, 0)),
        ],
        out_specs=pl.BlockSpec((BM, N_TAGS), lambda i: (i, 0)),
        out_shape=jax.ShapeDtypeStruct((bh, N_TAGS), jnp.float32),
    )(rows3, W13, b1, W2p, b2p)


def kernel(x, table, W1, b1, W2, b2):
    xi = x.astype(jnp.int32)
    W13 = W1.reshape(WINDOW, EMB, HIDDEN)
    W2p = jnp.pad(W2, ((0, 0), (0, 128 - N_TAGS)))
    b2p = jnp.pad(b2, (0, 128 - N_TAGS))
    b1r = b1.reshape(1, -1)
    b2r = b2p.reshape(1, -1)
    rows_list = []
    off = 0
    for bh in SPLITS:
        idx_h = xi[off:off + bh].T.reshape(-1)
        rows_list.append(_SC_GATHERS[bh](table, idx_h).reshape(WINDOW, bh, EMB))
        off += bh
    outs = [_mlp(bh, r, W13, b1r, W2p, b2r) for bh, r in zip(SPLITS, rows_list)]
    return jnp.concatenate(outs, axis=0)


# speedup vs baseline: 1.0224x; 1.0224x over previous
import functools

import jax
import jax.numpy as jnp
from jax import lax
from jax.experimental import pallas as pl
from jax.experimental.pallas import tpu as pltpu
from jax.experimental.pallas import tpu_sc as plsc

VOCAB = 1000000
EMB = 128
WINDOW = 5
HIDDEN = 256
N_TAGS = 50
BATCH = 16384

N_IDX = BATCH * WINDOW
NW = 32
B_PER_W = N_IDX // NW
CHUNK = 160
N_CHUNKS = B_PER_W // CHUNK
NBUF = 4
BM = 4096


def _sc_gather_body(table_hbm, idx_hbm, out_hbm, idx_v, *bufs):
    rows = bufs[:NBUF]
    sems = bufs[NBUF:]
    c = lax.axis_index("c")
    s = lax.axis_index("s")
    wid = s * 2 + c
    base = wid * B_PER_W
    pltpu.sync_copy(idx_hbm.at[pl.ds(base, B_PER_W)], idx_v)
    descs = [None] * NBUF
    for k in range(NBUF - 1):
        descs[k] = pltpu.async_copy(
            table_hbm.at[idx_v.at[pl.ds(k * CHUNK, CHUNK)]], rows[k], sems[k]
        )
    for k in range(N_CHUNKS):
        b = k % NBUF
        kn = k + NBUF - 1
        if kn < N_CHUNKS:
            bn = kn % NBUF
            descs[bn] = pltpu.async_copy(
                table_hbm.at[idx_v.at[pl.ds(kn * CHUNK, CHUNK)]],
                rows[bn],
                sems[bn],
            )
        descs[b].wait()
        pltpu.sync_copy(rows[b], out_hbm.at[pl.ds(base + k * CHUNK, CHUNK)])


@jax.jit
def _sc_gather(table, idx):
    mesh = plsc.VectorSubcoreMesh(core_axis_name="c", subcore_axis_name="s")
    run = pl.kernel(
        _sc_gather_body,
        mesh=mesh,
        out_type=jax.ShapeDtypeStruct((N_IDX, EMB), jnp.float32),
        scratch_types=(
            [pltpu.VMEM((B_PER_W,), jnp.int32)]
            + [pltpu.VMEM((CHUNK, EMB), jnp.float32) for _ in range(NBUF)]
            + [pltpu.SemaphoreType.DMA for _ in range(NBUF)]
        ),
    )
    return run(table, idx)


def _mlp_body(rows_ref, w1_ref, b1_ref, w2_ref, b2_ref, out_ref):
    acc = b1_ref[...] + jnp.dot(
        rows_ref[0], w1_ref[0], preferred_element_type=jnp.float32
    )
    for w in range(1, WINDOW):
        acc = acc + jnp.dot(
            rows_ref[w], w1_ref[w], preferred_element_type=jnp.float32
        )
    h = jnp.tanh(acc)
    out = jnp.dot(h, w2_ref[...], preferred_element_type=jnp.float32) + b2_ref[...]
    out_ref[...] = out[:, :N_TAGS]


@jax.jit
def _mlp(rows3, W13, b1, W2p, b2p):
    return pl.pallas_call(
        _mlp_body,
        grid=(BATCH // BM,),
        in_specs=[
            pl.BlockSpec((WINDOW, BM, EMB), lambda i: (0, i, 0)),
            pl.BlockSpec((WINDOW, EMB, HIDDEN), lambda i: (0, 0, 0)),
            pl.BlockSpec((1, HIDDEN), lambda i: (0, 0)),
            pl.BlockSpec((HIDDEN, 128), lambda i: (0, 0)),
            pl.BlockSpec((1, 128), lambda i: (0, 0)),
        ],
        out_specs=pl.BlockSpec((BM, N_TAGS), lambda i: (i, 0)),
        out_shape=jax.ShapeDtypeStruct((BATCH, N_TAGS), jnp.float32),
    )(rows3, W13, b1, W2p, b2p)


def kernel(x, table, W1, b1, W2, b2):
    idx = x.astype(jnp.int32).T.reshape(-1)
    rows = _sc_gather(table, idx)
    rows3 = rows.reshape(WINDOW, BATCH, EMB)
    W13 = W1.reshape(WINDOW, EMB, HIDDEN)
    W2p = jnp.pad(W2, ((0, 0), (0, 128 - N_TAGS)))
    b2p = jnp.pad(b2, (0, 128 - N_TAGS))
    return _mlp(rows3, W13, b1.reshape(1, -1), W2p, b2p.reshape(1, -1))
